# baseline (speedup 1.0000x reference)
import jax
import jax.numpy as jnp
from jax.experimental import pallas as pl

N_NODES = 28
IN_DIM = 512
OUT_DIM = 512


def _gcn_fused(inputs_ref, w_ref, o_ref):
    packed = inputs_ref[...]
    x = packed[:, :IN_DIM]
    support = packed[:, IN_DIM:]
    pre = jnp.dot(x, w_ref[...], preferred_element_type=jnp.float32)
    out = jnp.dot(support, pre, preferred_element_type=jnp.float32)
    o_ref[...] = jnp.maximum(out, 0.0)


def kernel(inputs, weight):
    return pl.pallas_call(
        _gcn_fused,
        out_shape=jax.ShapeDtypeStruct((N_NODES, OUT_DIM), jnp.float32),
    )(inputs, weight)
